# baseline (speedup 1.0000x reference)
import dataclasses
import functools

import jax
import jax.numpy as jnp
from jax import lax
from jax.experimental import pallas as pl
from jax.experimental.pallas import tpu as pltpu
from jax.experimental.pallas import tpu_sc as plsc

_SIDE = 128
_N = 2097152
_NW = 32
_CHUNK = 128
_PER_TILE = _N // _NW
_N_CHUNKS = _PER_TILE // _CHUNK


def _build_table(vox):
    v = jnp.transpose(vox, (1, 2, 3, 0))
    vx = jnp.stack(
        [v, jnp.concatenate([v[:, :, 1:], v[:, :, -1:]], axis=2)], axis=3
    )
    vy = jnp.stack(
        [vx, jnp.concatenate([vx[:, 1:], vx[:, -1:]], axis=1)], axis=3
    )
    vz = jnp.stack(
        [vy, jnp.concatenate([vy[1:], vy[-1:]], axis=0)], axis=3
    )
    return vz.reshape(_SIDE * _SIDE * _SIDE, 32)


def _compiler_params():
    cp = pltpu.CompilerParams()
    fields = pltpu.CompilerParams.__dataclass_fields__
    if "needs_layout_passes" in fields:
        cp = dataclasses.replace(cp, needs_layout_passes=False)
    if "use_tc_tiling_on_sc" in fields:
        cp = dataclasses.replace(cp, use_tc_tiling_on_sc=False)
    return cp


def _sc_interp(pos_flat, table, bias16):
    mesh = plsc.VectorSubcoreMesh(core_axis_name="c", subcore_axis_name="s")

    @functools.partial(
        pl.kernel,
        out_type=jax.ShapeDtypeStruct((_N * 4,), jnp.float32),
        mesh=mesh,
        compiler_params=_compiler_params(),
        scratch_types=[
            pltpu.VMEM((_CHUNK * 3,), jnp.float32),
            pltpu.VMEM((_CHUNK,), jnp.int32),
            pltpu.VMEM((_CHUNK, 32), jnp.float32),
            pltpu.VMEM((_CHUNK * 4,), jnp.float32),
            pltpu.VMEM((16,), jnp.float32),
        ],
    )
    def k(pos_hbm, tab_hbm, bias_hbm, out_hbm, pos_v, idx_v, rows_v, out_v,
          bias_v):
        wid = lax.axis_index("s") * 2 + lax.axis_index("c")
        base0 = wid * _PER_TILE
        pltpu.sync_copy(bias_hbm, bias_v)
        lane = lax.iota(jnp.int32, 16)
        b_splat = [
            plsc.load_gather(bias_v, [jnp.full((16,), c + 1, jnp.int32)])
            for c in range(4)
        ]
        half = jnp.float32(_SIDE // 2)
        top = jnp.float32(_SIDE - 1)

        def coords(g):
            rows = g * 16 + lane
            p3 = rows * 3
            x = plsc.load_gather(pos_v, [p3])
            y = plsc.load_gather(pos_v, [p3 + 1])
            z = plsc.load_gather(pos_v, [p3 + 2])
            ix = jnp.clip((x + 1.0) * half - 0.5, 0.0, top)
            iy = jnp.clip((y + 1.0) * half - 0.5, 0.0, top)
            iz = jnp.clip((z + 1.0) * half - 0.5, 0.0, top)
            ix0 = ix.astype(jnp.int32)
            iy0 = iy.astype(jnp.int32)
            iz0 = iz.astype(jnp.int32)
            fx = ix - ix0.astype(jnp.float32)
            fy = iy - iy0.astype(jnp.float32)
            fz = iz - iz0.astype(jnp.float32)
            return rows, ix0, iy0, iz0, fx, fy, fz

        @pl.loop(0, _N_CHUNKS)
        def _(ch):
            base = base0 + ch * _CHUNK
            pltpu.sync_copy(pos_hbm.at[pl.ds(base * 3, _CHUNK * 3)], pos_v)

            for g in range(_CHUNK // 16):
                _, ix0, iy0, iz0, _, _, _ = coords(g)
                lin = (iz0 * _SIDE + iy0) * _SIDE + ix0
                idx_v[pl.ds(g * 16, 16)] = lin

            pltpu.sync_copy(tab_hbm.at[idx_v], rows_v)

            for g in range(_CHUNK // 16):
                rows, _, _, _, fx, fy, fz = coords(g)
                gx = (1.0 - fx, fx)
                gy = (1.0 - fy, fy)
                gz = (1.0 - fz, fz)
                a = [[gz[zz] * gy[yy] for yy in range(2)] for zz in range(2)]
                w = [a[k8 >> 2][(k8 >> 1) & 1] * gx[k8 & 1] for k8 in range(8)]
                for c in range(4):
                    acc = b_splat[c]
                    for k8 in range(8):
                        val = plsc.load_gather(
                            rows_v,
                            [rows, jnp.full((16,), k8 * 4 + c, jnp.int32)],
                        )
                        acc = acc + w[k8] * val
                    plsc.store_scatter(out_v, [rows * 4 + c], acc)

            pltpu.sync_copy(out_v, out_hbm.at[pl.ds(base * 4, _CHUNK * 4)])

    return k(pos_flat, table, bias16)


def kernel(positions, voxels, bias):
    table = _build_table(voxels)
    pos_flat = positions.reshape(_N * 3)
    bias16 = jnp.pad(bias.reshape(4), (1, 11))
    out = _sc_interp(pos_flat, table, bias16)
    return out.reshape(_N, 4)


# speedup vs baseline: 1.0596x; 1.0596x over previous
import dataclasses
import functools

import jax
import jax.numpy as jnp
from jax import lax
from jax.experimental import pallas as pl
from jax.experimental.pallas import tpu as pltpu
from jax.experimental.pallas import tpu_sc as plsc

_SIDE = 128
_N = 2097152
_NW = 32
_CHUNK = 1024
_NSUB = 8
_GSUB = (_CHUNK // _NSUB) // 16
_PER_TILE = _N // _NW
_N_CHUNKS = _PER_TILE // _CHUNK


def _build_table(vox):
    v = jnp.transpose(vox, (1, 2, 3, 0))
    vx = jnp.stack(
        [v, jnp.concatenate([v[:, :, 1:], v[:, :, -1:]], axis=2)], axis=3
    )
    vy = jnp.stack(
        [vx, jnp.concatenate([vx[:, 1:], vx[:, -1:]], axis=1)], axis=3
    )
    vz = jnp.stack(
        [vy, jnp.concatenate([vy[1:], vy[-1:]], axis=0)], axis=3
    )
    return vz.reshape(_SIDE * _SIDE * _SIDE, 32)


def _compiler_params():
    cp = pltpu.CompilerParams()
    fields = pltpu.CompilerParams.__dataclass_fields__
    if "needs_layout_passes" in fields:
        cp = dataclasses.replace(cp, needs_layout_passes=False)
    if "use_tc_tiling_on_sc" in fields:
        cp = dataclasses.replace(cp, use_tc_tiling_on_sc=False)
    return cp


def _sc_interp(pos_flat, table, bias16):
    mesh = plsc.VectorSubcoreMesh(core_axis_name="c", subcore_axis_name="s")

    @functools.partial(
        pl.kernel,
        out_type=jax.ShapeDtypeStruct((_N * 4,), jnp.float32),
        mesh=mesh,
        compiler_params=_compiler_params(),
        scratch_types=[
            pltpu.VMEM((2 * _CHUNK * 3,), jnp.float32),
            pltpu.VMEM((2 * 3 * _CHUNK,), jnp.float32),
            pltpu.VMEM((2 * _NSUB, 128), jnp.int32),
            pltpu.VMEM((2 * _CHUNK, 32), jnp.float32),
            pltpu.VMEM((2 * _CHUNK * 4,), jnp.float32),
            pltpu.VMEM((16,), jnp.float32),
            pltpu.SemaphoreType.DMA,
            pltpu.SemaphoreType.DMA,
            pltpu.SemaphoreType.DMA,
            pltpu.SemaphoreType.DMA,
            pltpu.SemaphoreType.DMA,
        ],
    )
    def k(pos_hbm, tab_hbm, bias_hbm, out_hbm, pos_v, frac_v, idx_v, rows_v,
          out_v, bias_v, sem_pos, sem_ga, sem_gb, sem_oa, sem_ob):
        wid = lax.axis_index("s") * 2 + lax.axis_index("c")
        base0 = wid * _PER_TILE
        pltpu.sync_copy(bias_hbm, bias_v)
        lane = lax.iota(jnp.int32, 16)
        b_splat = [
            plsc.load_gather(bias_v, [jnp.full((16,), c + 1, jnp.int32)])
            for c in range(4)
        ]
        half = jnp.float32(_SIDE // 2)
        top = jnp.float32(_SIDE - 1)
        gsems = (sem_ga, sem_gb)
        osems = (sem_oa, sem_ob)

        def fire_pos(ch_next, par):
            pltpu.async_copy(
                pos_hbm.at[pl.ds((base0 + ch_next * _CHUNK) * 3, _CHUNK * 3)],
                pos_v.at[pl.ds(par * _CHUNK * 3, _CHUNK * 3)],
                sem_pos,
            )

        def drain_pos(par):
            pltpu.make_async_copy(
                pos_hbm.at[pl.ds(0, _CHUNK * 3)],
                pos_v.at[pl.ds(par * _CHUNK * 3, _CHUNK * 3)],
                sem_pos,
            ).wait()

        def index_phase(par):
            pb = par * _CHUNK * 3
            fb = par * 3 * _CHUNK
            sem_g = gsems[par]

            @pl.loop(0, _NSUB)
            def _(j):
                @pl.loop(0, _GSUB)
                def _(g2):
                    s0 = j * (_CHUNK // _NSUB) + g2 * 16
                    p3 = pb + s0 * 3 + lane * 3
                    x = plsc.load_gather(pos_v, [p3])
                    y = plsc.load_gather(pos_v, [p3 + 1])
                    z = plsc.load_gather(pos_v, [p3 + 2])
                    ix = jnp.clip((x + 1.0) * half - 0.5, 0.0, top)
                    iy = jnp.clip((y + 1.0) * half - 0.5, 0.0, top)
                    iz = jnp.clip((z + 1.0) * half - 0.5, 0.0, top)
                    ix0 = ix.astype(jnp.int32)
                    iy0 = iy.astype(jnp.int32)
                    iz0 = iz.astype(jnp.int32)
                    fx = ix - ix0.astype(jnp.float32)
                    fy = iy - iy0.astype(jnp.float32)
                    fz = iz - iz0.astype(jnp.float32)
                    lin = (iz0 * _SIDE + iy0) * _SIDE + ix0
                    idx_v[par * _NSUB + j, pl.ds(g2 * 16, 16)] = lin
                    frac_v[pl.ds(fb + s0, 16)] = fx
                    frac_v[pl.ds(fb + _CHUNK + s0, 16)] = fy
                    frac_v[pl.ds(fb + 2 * _CHUNK + s0, 16)] = fz

                pltpu.async_copy(
                    tab_hbm.at[idx_v.at[par * _NSUB + j]],
                    rows_v.at[pl.ds(par * _CHUNK + j * (_CHUNK // _NSUB),
                                    _CHUNK // _NSUB)],
                    sem_g,
                )

        def combine(c, par):
            sem_g = gsems[par]
            sem_o = osems[par]
            pltpu.make_async_copy(
                tab_hbm.at[pl.ds(0, _CHUNK)],
                rows_v.at[pl.ds(par * _CHUNK, _CHUNK)],
                sem_g,
            ).wait()

            @pl.when(c >= 2)
            def _():
                pltpu.make_async_copy(
                    out_v.at[pl.ds(par * _CHUNK * 4, _CHUNK * 4)],
                    out_hbm.at[pl.ds(0, _CHUNK * 4)],
                    sem_o,
                ).wait()

            fb = par * 3 * _CHUNK
            ob = par * _CHUNK * 4
            rb = par * _CHUNK

            @pl.loop(0, _CHUNK // 16)
            def _(g):
                s0 = g * 16
                fx = frac_v[pl.ds(fb + s0, 16)]
                fy = frac_v[pl.ds(fb + _CHUNK + s0, 16)]
                fz = frac_v[pl.ds(fb + 2 * _CHUNK + s0, 16)]
                rowvec = rb + s0 + lane
                gx = (1.0 - fx, fx)
                gy = (1.0 - fy, fy)
                gz = (1.0 - fz, fz)
                a = [[gz[zz] * gy[yy] for yy in range(2)] for zz in range(2)]
                w = [a[k8 >> 2][(k8 >> 1) & 1] * gx[k8 & 1] for k8 in range(8)]
                for c4 in range(4):
                    acc = b_splat[c4]
                    for k8 in range(8):
                        val = plsc.load_gather(
                            rows_v,
                            [rowvec, jnp.full((16,), k8 * 4 + c4, jnp.int32)],
                        )
                        acc = acc + w[k8] * val
                    plsc.store_scatter(
                        out_v, [ob + (s0 + lane) * 4 + c4], acc
                    )

            pltpu.async_copy(
                out_v.at[pl.ds(ob, _CHUNK * 4)],
                out_hbm.at[pl.ds((base0 + c * _CHUNK) * 4, _CHUNK * 4)],
                sem_o,
            )

        fire_pos(0, 0)

        @pl.loop(0, _N_CHUNKS // 2)
        def _(t):
            ch0 = t * 2
            ch1 = ch0 + 1
            drain_pos(0)
            fire_pos(ch1, 1)
            index_phase(0)

            @pl.when(ch0 > 0)
            def _():
                combine(ch0 - 1, 1)

            drain_pos(1)

            @pl.when(ch1 + 1 < _N_CHUNKS)
            def _():
                fire_pos(ch1 + 1, 0)

            index_phase(1)
            combine(ch0, 0)

        combine(_N_CHUNKS - 1, 1)
        pltpu.make_async_copy(
            out_v.at[pl.ds(0, _CHUNK * 4)],
            out_hbm.at[pl.ds(0, _CHUNK * 4)], sem_oa,
        ).wait()
        pltpu.make_async_copy(
            out_v.at[pl.ds(_CHUNK * 4, _CHUNK * 4)],
            out_hbm.at[pl.ds(0, _CHUNK * 4)], sem_ob,
        ).wait()

    return k(pos_flat, table, bias16)


def kernel(positions, voxels, bias):
    table = _build_table(voxels)
    pos_flat = positions.reshape(_N * 3)
    bias16 = jnp.pad(bias.reshape(4), (1, 11))
    out = _sc_interp(pos_flat, table, bias16)
    return out.reshape(_N, 4)


# speedup vs baseline: 3.7438x; 3.5332x over previous
import dataclasses
import functools

import jax
import jax.numpy as jnp
from jax import lax
from jax.experimental import pallas as pl
from jax.experimental.pallas import tpu as pltpu
from jax.experimental.pallas import tpu_sc as plsc

_SIDE = 128
_N = 2097152
_NW = 32
_CHUNK = 1024
_NSUB = 8
_GSUB = (_CHUNK // _NSUB) // 16
_PER_TILE = _N // _NW
_N_CHUNKS = _PER_TILE // _CHUNK
_ROWS_PER_TILE = _SIDE * _SIDE // _NW


def _compiler_params():
    cp = pltpu.CompilerParams()
    fields = pltpu.CompilerParams.__dataclass_fields__
    if "needs_layout_passes" in fields:
        cp = dataclasses.replace(cp, needs_layout_passes=False)
    if "use_tc_tiling_on_sc" in fields:
        cp = dataclasses.replace(cp, use_tc_tiling_on_sc=False)
    return cp


_MESH = plsc.VectorSubcoreMesh(core_axis_name="c", subcore_axis_name="s")


def _sc_build(vox2d):
    @functools.partial(
        pl.kernel,
        out_type=jax.ShapeDtypeStruct((_SIDE * _SIDE * _SIDE * 32,),
                                      jnp.float32),
        mesh=_MESH,
        compiler_params=_compiler_params(),
        scratch_types=[
            pltpu.VMEM((32, 128), jnp.float32),
            pltpu.VMEM((2 * 4096,), jnp.float32),
            pltpu.SemaphoreType.DMA,
            pltpu.SemaphoreType.DMA,
            pltpu.SemaphoreType.DMA,
        ],
    )
    def k(vox_hbm, tab_hbm, src_v, out_v, sem_in, sem_oa, sem_ob):
        wid = lax.axis_index("s") * 2 + lax.axis_index("c")
        row0 = wid * _ROWS_PER_TILE
        lane = lax.iota(jnp.int32, 16)
        osems = (sem_oa, sem_ob)
        rr_h, xx_h = [], []
        for h in range(2):
            m = h * 16 + lane
            zz = (m >> 4) & 1
            yy = (m >> 3) & 1
            xx = (m >> 2) & 1
            c = m & 3
            rr_h.append((zz * 2 + yy) * 4 + c)
            xx_h.append(xx)

        def fire_in(gr, par):
            z = gr >> 7
            y = gr & (_SIDE - 1)
            for zz in range(2):
                zc = jnp.minimum(z + zz, _SIDE - 1)
                for yy in range(2):
                    yc = jnp.minimum(y + yy, _SIDE - 1)
                    pltpu.async_copy(
                        vox_hbm.at[:, pl.ds((zc * _SIDE + yc) * _SIDE, _SIDE)],
                        src_v.at[pl.ds(par * 16 + (zz * 2 + yy) * 4, 4), :],
                        sem_in,
                    )

        def drain_in(par):
            for q in range(4):
                pltpu.make_async_copy(
                    vox_hbm.at[:, pl.ds(0, _SIDE)],
                    src_v.at[pl.ds(par * 16 + q * 4, 4), :],
                    sem_in,
                ).wait()

        def emit(gr, par):
            ob = par * 4096
            rows0 = par * 16 + rr_h[0]
            rows1 = par * 16 + rr_h[1]

            @pl.loop(0, _SIDE)
            def _(x):
                col0 = jnp.minimum(x + xx_h[0], _SIDE - 1)
                col1 = jnp.minimum(x + xx_h[1], _SIDE - 1)
                out_v[pl.ds(ob + x * 32, 16)] = plsc.load_gather(
                    src_v, [rows0, col0])
                out_v[pl.ds(ob + x * 32 + 16, 16)] = plsc.load_gather(
                    src_v, [rows1, col1])

            pltpu.async_copy(
                out_v.at[pl.ds(ob, 4096)],
                tab_hbm.at[pl.ds(gr * 4096, 4096)],
                osems[par],
            )

        def drain_out(par):
            pltpu.make_async_copy(
                out_v.at[pl.ds(par * 4096, 4096)],
                tab_hbm.at[pl.ds(0, 4096)],
                osems[par],
            ).wait()

        fire_in(row0, 0)

        @pl.loop(0, _ROWS_PER_TILE // 2)
        def _(t):
            r0 = row0 + t * 2
            drain_in(0)
            fire_in(r0 + 1, 1)

            @pl.when(t >= 1)
            def _():
                drain_out(0)

            emit(r0, 0)
            drain_in(1)

            @pl.when(t + 1 < _ROWS_PER_TILE // 2)
            def _():
                fire_in(r0 + 2, 0)

            @pl.when(t >= 1)
            def _():
                drain_out(1)

            emit(r0 + 1, 1)

        drain_out(0)
        drain_out(1)

    return k(vox2d)


def _sc_interp(px, py, pz, table, bias16):
    @functools.partial(
        pl.kernel,
        out_type=jax.ShapeDtypeStruct((_N * 4,), jnp.float32),
        mesh=_MESH,
        compiler_params=_compiler_params(),
        scratch_types=[
            pltpu.VMEM((2 * _CHUNK,), jnp.float32),
            pltpu.VMEM((2 * _CHUNK,), jnp.float32),
            pltpu.VMEM((2 * _CHUNK,), jnp.float32),
            pltpu.VMEM((2 * 3 * _CHUNK,), jnp.float32),
            pltpu.VMEM((2 * _NSUB, 128), jnp.int32),
            pltpu.VMEM((2 * _CHUNK, 32), jnp.float32),
            pltpu.VMEM((2 * _CHUNK * 4,), jnp.float32),
            pltpu.VMEM((16,), jnp.float32),
            pltpu.SemaphoreType.DMA,
            pltpu.SemaphoreType.DMA,
            pltpu.SemaphoreType.DMA,
            pltpu.SemaphoreType.DMA,
            pltpu.SemaphoreType.DMA,
        ],
    )
    def k(px_hbm, py_hbm, pz_hbm, tab_hbm, bias_hbm, out_hbm,
          px_v, py_v, pz_v, frac_v, idx_v, rows_v, out_v, bias_v,
          sem_pos, sem_ga, sem_gb, sem_oa, sem_ob):
        wid = lax.axis_index("s") * 2 + lax.axis_index("c")
        base0 = wid * _PER_TILE
        pltpu.sync_copy(bias_hbm, bias_v)
        lane = lax.iota(jnp.int32, 16)
        b_splat = [
            plsc.load_gather(bias_v, [jnp.full((16,), c + 1, jnp.int32)])
            for c in range(4)
        ]
        half = jnp.float32(_SIDE // 2)
        top = jnp.float32(_SIDE - 1)
        gsems = (sem_ga, sem_gb)
        osems = (sem_oa, sem_ob)
        coord_bufs = ((px_hbm, px_v), (py_hbm, py_v), (pz_hbm, pz_v))

        def fire_pos(ch_next, par):
            base = base0 + ch_next * _CHUNK
            for hbm, vmem in coord_bufs:
                pltpu.async_copy(
                    hbm.at[pl.ds(base, _CHUNK)],
                    vmem.at[pl.ds(par * _CHUNK, _CHUNK)],
                    sem_pos,
                )

        def drain_pos(par):
            for hbm, vmem in coord_bufs:
                pltpu.make_async_copy(
                    hbm.at[pl.ds(0, _CHUNK)],
                    vmem.at[pl.ds(par * _CHUNK, _CHUNK)],
                    sem_pos,
                ).wait()

        def index_phase(par):
            pb = par * _CHUNK
            fb = par * 3 * _CHUNK
            sem_g = gsems[par]

            @pl.loop(0, _NSUB)
            def _(j):
                @pl.loop(0, _GSUB)
                def _(g2):
                    s0 = j * (_CHUNK // _NSUB) + g2 * 16
                    x = px_v[pl.ds(pb + s0, 16)]
                    y = py_v[pl.ds(pb + s0, 16)]
                    z = pz_v[pl.ds(pb + s0, 16)]
                    ix = jnp.clip((x + 1.0) * half - 0.5, 0.0, top)
                    iy = jnp.clip((y + 1.0) * half - 0.5, 0.0, top)
                    iz = jnp.clip((z + 1.0) * half - 0.5, 0.0, top)
                    ix0 = ix.astype(jnp.int32)
                    iy0 = iy.astype(jnp.int32)
                    iz0 = iz.astype(jnp.int32)
                    fx = ix - ix0.astype(jnp.float32)
                    fy = iy - iy0.astype(jnp.float32)
                    fz = iz - iz0.astype(jnp.float32)
                    lin = (iz0 * _SIDE + iy0) * _SIDE + ix0
                    idx_v[par * _NSUB + j, pl.ds(g2 * 16, 16)] = lin
                    frac_v[pl.ds(fb + s0, 16)] = fx
                    frac_v[pl.ds(fb + _CHUNK + s0, 16)] = fy
                    frac_v[pl.ds(fb + 2 * _CHUNK + s0, 16)] = fz

                pltpu.async_copy(
                    tab_hbm.at[idx_v.at[par * _NSUB + j]],
                    rows_v.at[pl.ds(par * _CHUNK + j * (_CHUNK // _NSUB),
                                    _CHUNK // _NSUB)],
                    sem_g,
                )

        def combine(c, par):
            sem_g = gsems[par]
            sem_o = osems[par]
            pltpu.make_async_copy(
                tab_hbm.at[pl.ds(0, _CHUNK)],
                rows_v.at[pl.ds(par * _CHUNK, _CHUNK)],
                sem_g,
            ).wait()

            @pl.when(c >= 2)
            def _():
                pltpu.make_async_copy(
                    out_v.at[pl.ds(par * _CHUNK * 4, _CHUNK * 4)],
                    out_hbm.at[pl.ds(0, _CHUNK * 4)],
                    sem_o,
                ).wait()

            fb = par * 3 * _CHUNK
            ob = par * _CHUNK * 4
            rb = par * _CHUNK

            @pl.loop(0, _CHUNK // 16)
            def _(g):
                s0 = g * 16
                fx = frac_v[pl.ds(fb + s0, 16)]
                fy = frac_v[pl.ds(fb + _CHUNK + s0, 16)]
                fz = frac_v[pl.ds(fb + 2 * _CHUNK + s0, 16)]
                rowvec = rb + s0 + lane
                gx = (1.0 - fx, fx)
                gy = (1.0 - fy, fy)
                gz = (1.0 - fz, fz)
                a = [[gz[zz] * gy[yy] for yy in range(2)] for zz in range(2)]
                w = [a[k8 >> 2][(k8 >> 1) & 1] * gx[k8 & 1] for k8 in range(8)]
                for c4 in range(4):
                    acc = b_splat[c4]
                    for k8 in range(8):
                        val = plsc.load_gather(
                            rows_v,
                            [rowvec, jnp.full((16,), k8 * 4 + c4, jnp.int32)],
                        )
                        acc = acc + w[k8] * val
                    plsc.store_scatter(
                        out_v, [ob + (s0 + lane) * 4 + c4], acc
                    )

            pltpu.async_copy(
                out_v.at[pl.ds(ob, _CHUNK * 4)],
                out_hbm.at[pl.ds((base0 + c * _CHUNK) * 4, _CHUNK * 4)],
                sem_o,
            )

        fire_pos(0, 0)

        @pl.loop(0, _N_CHUNKS // 2)
        def _(t):
            ch0 = t * 2
            ch1 = ch0 + 1
            drain_pos(0)
            fire_pos(ch1, 1)
            index_phase(0)

            @pl.when(ch0 > 0)
            def _():
                combine(ch0 - 1, 1)

            drain_pos(1)

            @pl.when(ch1 + 1 < _N_CHUNKS)
            def _():
                fire_pos(ch1 + 1, 0)

            index_phase(1)
            combine(ch0, 0)

        combine(_N_CHUNKS - 1, 1)
        pltpu.make_async_copy(
            out_v.at[pl.ds(0, _CHUNK * 4)],
            out_hbm.at[pl.ds(0, _CHUNK * 4)], sem_oa,
        ).wait()
        pltpu.make_async_copy(
            out_v.at[pl.ds(_CHUNK * 4, _CHUNK * 4)],
            out_hbm.at[pl.ds(0, _CHUNK * 4)], sem_ob,
        ).wait()

    return k(px, py, pz, table, bias16)


def kernel(positions, voxels, bias):
    vox2d = voxels.reshape(4, _SIDE * _SIDE * _SIDE)
    table = _sc_build(vox2d).reshape(_SIDE * _SIDE * _SIDE, 32)
    px = positions[:, 0]
    py = positions[:, 1]
    pz = positions[:, 2]
    bias16 = jnp.pad(bias.reshape(4), (1, 11))
    out = _sc_interp(px, py, pz, table, bias16)
    return out.reshape(_N, 4)


# speedup vs baseline: 5.5396x; 1.4797x over previous
import dataclasses
import functools

import jax
import jax.numpy as jnp
from jax import lax
from jax.experimental import pallas as pl
from jax.experimental.pallas import tpu as pltpu
from jax.experimental.pallas import tpu_sc as plsc

_SIDE = 128
_N = 2097152
_NW = 32
_CHUNK = 1024
_NSUB = 8
_GSUB = (_CHUNK // _NSUB) // 16
_PER_TILE = _N // _NW
_N_CHUNKS = _PER_TILE // _CHUNK
_ROWS_PER_TILE = _SIDE * _SIDE // _NW


def _compiler_params():
    cp = pltpu.CompilerParams()
    fields = pltpu.CompilerParams.__dataclass_fields__
    if "needs_layout_passes" in fields:
        cp = dataclasses.replace(cp, needs_layout_passes=False)
    if "use_tc_tiling_on_sc" in fields:
        cp = dataclasses.replace(cp, use_tc_tiling_on_sc=False)
    return cp


_MESH = plsc.VectorSubcoreMesh(core_axis_name="c", subcore_axis_name="s")


def _sc_build(vox2d):
    @functools.partial(
        pl.kernel,
        out_type=jax.ShapeDtypeStruct((_SIDE * _SIDE * _SIDE * 32,),
                                      jnp.float32),
        mesh=_MESH,
        compiler_params=_compiler_params(),
        scratch_types=[
            pltpu.VMEM((32, 128), jnp.float32),
            pltpu.VMEM((2 * 4096,), jnp.float32),
            pltpu.SemaphoreType.DMA,
            pltpu.SemaphoreType.DMA,
            pltpu.SemaphoreType.DMA,
        ],
    )
    def k(vox_hbm, tab_hbm, src_v, out_v, sem_in, sem_oa, sem_ob):
        wid = lax.axis_index("s") * 2 + lax.axis_index("c")
        row0 = wid * _ROWS_PER_TILE
        lane = lax.iota(jnp.int32, 16)
        osems = (sem_oa, sem_ob)
        rr_h, xx_h = [], []
        for h in range(2):
            m = h * 16 + lane
            zz = (m >> 4) & 1
            yy = (m >> 3) & 1
            xx = (m >> 2) & 1
            c = m & 3
            rr_h.append((zz * 2 + yy) * 4 + c)
            xx_h.append(xx)

        def fire_in(gr, par):
            z = gr >> 7
            y = gr & (_SIDE - 1)
            for zz in range(2):
                zc = jnp.minimum(z + zz, _SIDE - 1)
                for yy in range(2):
                    yc = jnp.minimum(y + yy, _SIDE - 1)
                    pltpu.async_copy(
                        vox_hbm.at[:, pl.ds((zc * _SIDE + yc) * _SIDE, _SIDE)],
                        src_v.at[pl.ds(par * 16 + (zz * 2 + yy) * 4, 4), :],
                        sem_in,
                    )

        def drain_in(par):
            for q in range(4):
                pltpu.make_async_copy(
                    vox_hbm.at[:, pl.ds(0, _SIDE)],
                    src_v.at[pl.ds(par * 16 + q * 4, 4), :],
                    sem_in,
                ).wait()

        def emit(gr, par):
            ob = par * 4096
            rows0 = par * 16 + rr_h[0]
            rows1 = par * 16 + rr_h[1]

            for x in range(_SIDE):
                col0 = jnp.minimum(x + xx_h[0], _SIDE - 1)
                col1 = jnp.minimum(x + xx_h[1], _SIDE - 1)
                out_v[pl.ds(ob + x * 32, 16)] = plsc.load_gather(
                    src_v, [rows0, col0])
                out_v[pl.ds(ob + x * 32 + 16, 16)] = plsc.load_gather(
                    src_v, [rows1, col1])

            pltpu.async_copy(
                out_v.at[pl.ds(ob, 4096)],
                tab_hbm.at[pl.ds(gr * 4096, 4096)],
                osems[par],
            )

        def drain_out(par):
            pltpu.make_async_copy(
                out_v.at[pl.ds(par * 4096, 4096)],
                tab_hbm.at[pl.ds(0, 4096)],
                osems[par],
            ).wait()

        fire_in(row0, 0)

        @pl.loop(0, _ROWS_PER_TILE // 2)
        def _(t):
            r0 = row0 + t * 2
            drain_in(0)
            fire_in(r0 + 1, 1)

            @pl.when(t >= 1)
            def _():
                drain_out(0)

            emit(r0, 0)
            drain_in(1)

            @pl.when(t + 1 < _ROWS_PER_TILE // 2)
            def _():
                fire_in(r0 + 2, 0)

            @pl.when(t >= 1)
            def _():
                drain_out(1)

            emit(r0 + 1, 1)

        drain_out(0)
        drain_out(1)

    return k(vox2d)


def _sc_interp(px, py, pz, table, bias16):
    @functools.partial(
        pl.kernel,
        out_type=jax.ShapeDtypeStruct((_N * 4,), jnp.float32),
        mesh=_MESH,
        compiler_params=_compiler_params(),
        scratch_types=[
            pltpu.VMEM((2 * _CHUNK,), jnp.float32),
            pltpu.VMEM((2 * _CHUNK,), jnp.float32),
            pltpu.VMEM((2 * _CHUNK,), jnp.float32),
            pltpu.VMEM((2 * 3 * _CHUNK,), jnp.float32),
            pltpu.VMEM((2 * _NSUB, 128), jnp.int32),
            pltpu.VMEM((2 * _CHUNK, 32), jnp.float32),
            pltpu.VMEM((2 * _CHUNK * 4,), jnp.float32),
            pltpu.VMEM((16,), jnp.float32),
            pltpu.SemaphoreType.DMA,
            pltpu.SemaphoreType.DMA,
            pltpu.SemaphoreType.DMA,
            pltpu.SemaphoreType.DMA,
            pltpu.SemaphoreType.DMA,
        ],
    )
    def k(px_hbm, py_hbm, pz_hbm, tab_hbm, bias_hbm, out_hbm,
          px_v, py_v, pz_v, frac_v, idx_v, rows_v, out_v, bias_v,
          sem_pos, sem_ga, sem_gb, sem_oa, sem_ob):
        wid = lax.axis_index("s") * 2 + lax.axis_index("c")
        base0 = wid * _PER_TILE
        pltpu.sync_copy(bias_hbm, bias_v)
        lane = lax.iota(jnp.int32, 16)
        b_splat = [
            plsc.load_gather(bias_v, [jnp.full((16,), c + 1, jnp.int32)])
            for c in range(4)
        ]
        half = jnp.float32(_SIDE // 2)
        top = jnp.float32(_SIDE - 1)
        gsems = (sem_ga, sem_gb)
        osems = (sem_oa, sem_ob)
        coord_bufs = ((px_hbm, px_v), (py_hbm, py_v), (pz_hbm, pz_v))

        def fire_pos(ch_next, par):
            base = base0 + ch_next * _CHUNK
            for hbm, vmem in coord_bufs:
                pltpu.async_copy(
                    hbm.at[pl.ds(base, _CHUNK)],
                    vmem.at[pl.ds(par * _CHUNK, _CHUNK)],
                    sem_pos,
                )

        def drain_pos(par):
            for hbm, vmem in coord_bufs:
                pltpu.make_async_copy(
                    hbm.at[pl.ds(0, _CHUNK)],
                    vmem.at[pl.ds(par * _CHUNK, _CHUNK)],
                    sem_pos,
                ).wait()

        def index_phase(par):
            pb = par * _CHUNK
            fb = par * 3 * _CHUNK
            sem_g = gsems[par]

            @pl.loop(0, _NSUB)
            def _(j):
                @pl.loop(0, _GSUB)
                def _(g2):
                    s0 = j * (_CHUNK // _NSUB) + g2 * 16
                    x = px_v[pl.ds(pb + s0, 16)]
                    y = py_v[pl.ds(pb + s0, 16)]
                    z = pz_v[pl.ds(pb + s0, 16)]
                    ix = jnp.clip((x + 1.0) * half - 0.5, 0.0, top)
                    iy = jnp.clip((y + 1.0) * half - 0.5, 0.0, top)
                    iz = jnp.clip((z + 1.0) * half - 0.5, 0.0, top)
                    ix0 = ix.astype(jnp.int32)
                    iy0 = iy.astype(jnp.int32)
                    iz0 = iz.astype(jnp.int32)
                    fx = ix - ix0.astype(jnp.float32)
                    fy = iy - iy0.astype(jnp.float32)
                    fz = iz - iz0.astype(jnp.float32)
                    lin = (iz0 * _SIDE + iy0) * _SIDE + ix0
                    idx_v[par * _NSUB + j, pl.ds(g2 * 16, 16)] = lin
                    frac_v[pl.ds(fb + s0, 16)] = fx
                    frac_v[pl.ds(fb + _CHUNK + s0, 16)] = fy
                    frac_v[pl.ds(fb + 2 * _CHUNK + s0, 16)] = fz

                pltpu.async_copy(
                    tab_hbm.at[idx_v.at[par * _NSUB + j]],
                    rows_v.at[pl.ds(par * _CHUNK + j * (_CHUNK // _NSUB),
                                    _CHUNK // _NSUB)],
                    sem_g,
                )

        def combine(c, par):
            sem_g = gsems[par]
            sem_o = osems[par]
            pltpu.make_async_copy(
                tab_hbm.at[pl.ds(0, _CHUNK)],
                rows_v.at[pl.ds(par * _CHUNK, _CHUNK)],
                sem_g,
            ).wait()

            @pl.when(c >= 2)
            def _():
                pltpu.make_async_copy(
                    out_v.at[pl.ds(par * _CHUNK * 4, _CHUNK * 4)],
                    out_hbm.at[pl.ds(0, _CHUNK * 4)],
                    sem_o,
                ).wait()

            fb = par * 3 * _CHUNK
            ob = par * _CHUNK * 4
            rb = par * _CHUNK

            @pl.loop(0, _CHUNK // 16)
            def _(g):
                s0 = g * 16
                fx = frac_v[pl.ds(fb + s0, 16)]
                fy = frac_v[pl.ds(fb + _CHUNK + s0, 16)]
                fz = frac_v[pl.ds(fb + 2 * _CHUNK + s0, 16)]
                rowvec = rb + s0 + lane
                gx = (1.0 - fx, fx)
                gy = (1.0 - fy, fy)
                gz = (1.0 - fz, fz)
                a = [[gz[zz] * gy[yy] for yy in range(2)] for zz in range(2)]
                w = [a[k8 >> 2][(k8 >> 1) & 1] * gx[k8 & 1] for k8 in range(8)]
                blk = ob + (s0 >> 7) * 512 + (s0 & 127)
                for c4 in range(4):
                    acc = b_splat[c4]
                    for k8 in range(8):
                        val = plsc.load_gather(
                            rows_v,
                            [rowvec, jnp.full((16,), k8 * 4 + c4, jnp.int32)],
                        )
                        acc = acc + w[k8] * val
                    out_v[pl.ds(blk + c4 * 128, 16)] = acc

            pltpu.async_copy(
                out_v.at[pl.ds(ob, _CHUNK * 4)],
                out_hbm.at[pl.ds((base0 + c * _CHUNK) * 4, _CHUNK * 4)],
                sem_o,
            )

        fire_pos(0, 0)

        @pl.loop(0, _N_CHUNKS // 2)
        def _(t):
            ch0 = t * 2
            ch1 = ch0 + 1
            drain_pos(0)
            fire_pos(ch1, 1)
            index_phase(0)

            @pl.when(ch0 > 0)
            def _():
                combine(ch0 - 1, 1)

            drain_pos(1)

            @pl.when(ch1 + 1 < _N_CHUNKS)
            def _():
                fire_pos(ch1 + 1, 0)

            index_phase(1)
            combine(ch0, 0)

        combine(_N_CHUNKS - 1, 1)
        pltpu.make_async_copy(
            out_v.at[pl.ds(0, _CHUNK * 4)],
            out_hbm.at[pl.ds(0, _CHUNK * 4)], sem_oa,
        ).wait()
        pltpu.make_async_copy(
            out_v.at[pl.ds(_CHUNK * 4, _CHUNK * 4)],
            out_hbm.at[pl.ds(0, _CHUNK * 4)], sem_ob,
        ).wait()

    return k(px, py, pz, table, bias16)


def kernel(positions, voxels, bias):
    vox2d = voxels.reshape(4, _SIDE * _SIDE * _SIDE)
    table = _sc_build(vox2d).reshape(_SIDE * _SIDE * _SIDE, 32)
    px = positions[:, 0]
    py = positions[:, 1]
    pz = positions[:, 2]
    bias16 = jnp.pad(bias.reshape(4), (1, 11))
    out = _sc_interp(px, py, pz, table, bias16)
    g = out.reshape(_N // 128, 4, 128)
    return jnp.swapaxes(g, 1, 2).reshape(_N, 4)


# speedup vs baseline: 6.2688x; 1.1316x over previous
import dataclasses
import functools

import jax
import jax.numpy as jnp
from jax import lax
from jax.experimental import pallas as pl
from jax.experimental.pallas import tpu as pltpu
from jax.experimental.pallas import tpu_sc as plsc

_SIDE = 128
_N = 2097152
_NW = 32
_CHUNK = 1024
_NSUB = 8
_GSUB = (_CHUNK // _NSUB) // 16
_PER_TILE = _N // _NW
_N_CHUNKS = _PER_TILE // _CHUNK
_ROWS_PER_TILE = _SIDE * _SIDE // _NW


def _compiler_params():
    cp = pltpu.CompilerParams()
    fields = pltpu.CompilerParams.__dataclass_fields__
    if "needs_layout_passes" in fields:
        cp = dataclasses.replace(cp, needs_layout_passes=False)
    if "use_tc_tiling_on_sc" in fields:
        cp = dataclasses.replace(cp, use_tc_tiling_on_sc=False)
    return cp


_MESH = plsc.VectorSubcoreMesh(core_axis_name="c", subcore_axis_name="s")


def _sc_build(vox2d):
    @functools.partial(
        pl.kernel,
        out_type=jax.ShapeDtypeStruct((_SIDE * _SIDE * _SIDE * 32,),
                                      jnp.float32),
        mesh=_MESH,
        compiler_params=_compiler_params(),
        scratch_types=[
            pltpu.VMEM((32, 128), jnp.float32),
            pltpu.VMEM((2 * 4096,), jnp.float32),
            pltpu.SemaphoreType.DMA,
            pltpu.SemaphoreType.DMA,
            pltpu.SemaphoreType.DMA,
        ],
    )
    def k(vox_hbm, tab_hbm, src_v, out_v, sem_in, sem_oa, sem_ob):
        wid = lax.axis_index("s") * 2 + lax.axis_index("c")
        row0 = wid * _ROWS_PER_TILE
        lane = lax.iota(jnp.int32, 16)
        osems = (sem_oa, sem_ob)
        rr_h, xx_h = [], []
        for h in range(2):
            m = h * 16 + lane
            zz = (m >> 4) & 1
            yy = (m >> 3) & 1
            xx = (m >> 2) & 1
            c = m & 3
            rr_h.append((zz * 2 + yy) * 4 + c)
            xx_h.append(xx)

        def fire_in(gr, par):
            z = gr >> 7
            y = gr & (_SIDE - 1)
            for zz in range(2):
                zc = jnp.minimum(z + zz, _SIDE - 1)
                for yy in range(2):
                    yc = jnp.minimum(y + yy, _SIDE - 1)
                    pltpu.async_copy(
                        vox_hbm.at[:, pl.ds((zc * _SIDE + yc) * _SIDE, _SIDE)],
                        src_v.at[pl.ds(par * 16 + (zz * 2 + yy) * 4, 4), :],
                        sem_in,
                    )

        def drain_in(par):
            for q in range(4):
                pltpu.make_async_copy(
                    vox_hbm.at[:, pl.ds(0, _SIDE)],
                    src_v.at[pl.ds(par * 16 + q * 4, 4), :],
                    sem_in,
                ).wait()

        def emit(gr, par):
            ob = par * 4096
            rows0 = par * 16 + rr_h[0]
            rows1 = par * 16 + rr_h[1]

            @pl.loop(0, _SIDE)
            def _(x):
                col0 = jnp.minimum(x + xx_h[0], _SIDE - 1)
                col1 = jnp.minimum(x + xx_h[1], _SIDE - 1)
                out_v[pl.ds(ob + x * 32, 16)] = plsc.load_gather(
                    src_v, [rows0, col0])
                out_v[pl.ds(ob + x * 32 + 16, 16)] = plsc.load_gather(
                    src_v, [rows1, col1])

            pltpu.async_copy(
                out_v.at[pl.ds(ob, 4096)],
                tab_hbm.at[pl.ds(gr * 4096, 4096)],
                osems[par],
            )

        def drain_out(par):
            pltpu.make_async_copy(
                out_v.at[pl.ds(par * 4096, 4096)],
                tab_hbm.at[pl.ds(0, 4096)],
                osems[par],
            ).wait()

        fire_in(row0, 0)

        @pl.loop(0, _ROWS_PER_TILE // 2)
        def _(t):
            r0 = row0 + t * 2
            drain_in(0)
            fire_in(r0 + 1, 1)

            @pl.when(t >= 1)
            def _():
                drain_out(0)

            emit(r0, 0)
            drain_in(1)

            @pl.when(t + 1 < _ROWS_PER_TILE // 2)
            def _():
                fire_in(r0 + 2, 0)

            @pl.when(t >= 1)
            def _():
                drain_out(1)

            emit(r0 + 1, 1)

        drain_out(0)
        drain_out(1)

    return k(vox2d)


def _sc_interp(px, py, pz, table, bias16):
    @functools.partial(
        pl.kernel,
        out_type=jax.ShapeDtypeStruct((_N * 4,), jnp.float32),
        mesh=_MESH,
        compiler_params=_compiler_params(),
        scratch_types=[
            pltpu.VMEM((2 * _CHUNK,), jnp.float32),
            pltpu.VMEM((2 * _CHUNK,), jnp.float32),
            pltpu.VMEM((2 * _CHUNK,), jnp.float32),
            pltpu.VMEM((2 * 3 * _CHUNK,), jnp.float32),
            pltpu.VMEM((2 * _NSUB, 128), jnp.int32),
            pltpu.VMEM((2 * _CHUNK, 32), jnp.float32),
            pltpu.VMEM((2 * _CHUNK * 4,), jnp.float32),
            pltpu.VMEM((16,), jnp.float32),
            pltpu.SemaphoreType.DMA,
            pltpu.SemaphoreType.DMA,
            pltpu.SemaphoreType.DMA,
            pltpu.SemaphoreType.DMA,
            pltpu.SemaphoreType.DMA,
        ],
    )
    def k(px_hbm, py_hbm, pz_hbm, tab_hbm, bias_hbm, out_hbm,
          px_v, py_v, pz_v, frac_v, idx_v, rows_v, out_v, bias_v,
          sem_pos, sem_ga, sem_gb, sem_oa, sem_ob):
        wid = lax.axis_index("s") * 2 + lax.axis_index("c")
        base0 = wid * _PER_TILE
        pltpu.sync_copy(bias_hbm, bias_v)
        lane = lax.iota(jnp.int32, 16)
        b_splat = [
            plsc.load_gather(bias_v, [jnp.full((16,), c + 1, jnp.int32)])
            for c in range(4)
        ]
        half = jnp.float32(_SIDE // 2)
        top = jnp.float32(_SIDE - 1)
        gsems = (sem_ga, sem_gb)
        osems = (sem_oa, sem_ob)
        coord_bufs = ((px_hbm, px_v), (py_hbm, py_v), (pz_hbm, pz_v))

        def fire_pos(ch_next, par):
            base = base0 + ch_next * _CHUNK
            for hbm, vmem in coord_bufs:
                pltpu.async_copy(
                    hbm.at[pl.ds(base, _CHUNK)],
                    vmem.at[pl.ds(par * _CHUNK, _CHUNK)],
                    sem_pos,
                )

        def drain_pos(par):
            for hbm, vmem in coord_bufs:
                pltpu.make_async_copy(
                    hbm.at[pl.ds(0, _CHUNK)],
                    vmem.at[pl.ds(par * _CHUNK, _CHUNK)],
                    sem_pos,
                ).wait()

        def index_phase(par):
            pb = par * _CHUNK
            fb = par * 3 * _CHUNK
            sem_g = gsems[par]

            @pl.loop(0, _NSUB)
            def _(j):
                @pl.loop(0, _GSUB)
                def _(g2):
                    s0 = j * (_CHUNK // _NSUB) + g2 * 16
                    x = px_v[pl.ds(pb + s0, 16)]
                    y = py_v[pl.ds(pb + s0, 16)]
                    z = pz_v[pl.ds(pb + s0, 16)]
                    ix = jnp.clip((x + 1.0) * half - 0.5, 0.0, top)
                    iy = jnp.clip((y + 1.0) * half - 0.5, 0.0, top)
                    iz = jnp.clip((z + 1.0) * half - 0.5, 0.0, top)
                    ix0 = ix.astype(jnp.int32)
                    iy0 = iy.astype(jnp.int32)
                    iz0 = iz.astype(jnp.int32)
                    fx = ix - ix0.astype(jnp.float32)
                    fy = iy - iy0.astype(jnp.float32)
                    fz = iz - iz0.astype(jnp.float32)
                    lin = (iz0 * _SIDE + iy0) * _SIDE + ix0
                    idx_v[par * _NSUB + j, pl.ds(g2 * 16, 16)] = lin
                    frac_v[pl.ds(fb + s0, 16)] = fx
                    frac_v[pl.ds(fb + _CHUNK + s0, 16)] = fy
                    frac_v[pl.ds(fb + 2 * _CHUNK + s0, 16)] = fz

                pltpu.async_copy(
                    tab_hbm.at[idx_v.at[par * _NSUB + j]],
                    rows_v.at[pl.ds(par * _CHUNK + j * (_CHUNK // _NSUB),
                                    _CHUNK // _NSUB)],
                    sem_g,
                )

        def combine(c, par):
            sem_g = gsems[par]
            sem_o = osems[par]
            pltpu.make_async_copy(
                tab_hbm.at[pl.ds(0, _CHUNK)],
                rows_v.at[pl.ds(par * _CHUNK, _CHUNK)],
                sem_g,
            ).wait()

            @pl.when(c >= 2)
            def _():
                pltpu.make_async_copy(
                    out_v.at[pl.ds(par * _CHUNK * 4, _CHUNK * 4)],
                    out_hbm.at[pl.ds(0, _CHUNK * 4)],
                    sem_o,
                ).wait()

            fb = par * 3 * _CHUNK
            ob = par * _CHUNK * 4
            rb = par * _CHUNK

            @pl.loop(0, _CHUNK // 16)
            def _(g):
                s0 = g * 16
                fx = frac_v[pl.ds(fb + s0, 16)]
                fy = frac_v[pl.ds(fb + _CHUNK + s0, 16)]
                fz = frac_v[pl.ds(fb + 2 * _CHUNK + s0, 16)]
                rowvec = rb + s0 + lane
                gx = (1.0 - fx, fx)
                gy = (1.0 - fy, fy)
                gz = (1.0 - fz, fz)
                a = [[gz[zz] * gy[yy] for yy in range(2)] for zz in range(2)]
                w = [a[k8 >> 2][(k8 >> 1) & 1] * gx[k8 & 1] for k8 in range(8)]
                blk = ob + (s0 >> 7) * 512 + (s0 & 127)
                for c4 in range(4):
                    acc = b_splat[c4]
                    for k8 in range(8):
                        val = plsc.load_gather(
                            rows_v,
                            [rowvec, jnp.full((16,), k8 * 4 + c4, jnp.int32)],
                        )
                        acc = acc + w[k8] * val
                    out_v[pl.ds(blk + c4 * 128, 16)] = acc

            pltpu.async_copy(
                out_v.at[pl.ds(ob, _CHUNK * 4)],
                out_hbm.at[pl.ds((base0 + c * _CHUNK) * 4, _CHUNK * 4)],
                sem_o,
            )

        fire_pos(0, 0)

        @pl.loop(0, _N_CHUNKS // 2)
        def _(t):
            ch0 = t * 2
            ch1 = ch0 + 1
            drain_pos(0)
            fire_pos(ch1, 1)
            index_phase(0)

            @pl.when(ch0 > 0)
            def _():
                combine(ch0 - 1, 1)

            drain_pos(1)

            @pl.when(ch1 + 1 < _N_CHUNKS)
            def _():
                fire_pos(ch1 + 1, 0)

            index_phase(1)
            combine(ch0, 0)

        combine(_N_CHUNKS - 1, 1)
        pltpu.make_async_copy(
            out_v.at[pl.ds(0, _CHUNK * 4)],
            out_hbm.at[pl.ds(0, _CHUNK * 4)], sem_oa,
        ).wait()
        pltpu.make_async_copy(
            out_v.at[pl.ds(_CHUNK * 4, _CHUNK * 4)],
            out_hbm.at[pl.ds(0, _CHUNK * 4)], sem_ob,
        ).wait()

    return k(px, py, pz, table, bias16)


def kernel(positions, voxels, bias):
    vox2d = voxels.reshape(4, _SIDE * _SIDE * _SIDE)
    table = _sc_build(vox2d).reshape(_SIDE * _SIDE * _SIDE, 32)
    px = positions[:, 0]
    py = positions[:, 1]
    pz = positions[:, 2]
    bias16 = jnp.pad(bias.reshape(4), (1, 11))
    out = _sc_interp(px, py, pz, table, bias16)
    g = out.reshape(_N // 128, 4, 128)
    return jnp.swapaxes(g, 1, 2).reshape(_N, 4)
